# Initial kernel scaffold; baseline (speedup 1.0000x reference)
import jax
import jax.numpy as jnp
from jax.experimental import pallas as pl


def kernel(point_features, view_features, superpoint_ids, valid_mask, pp_W, pp_b, pp_g, pp_be, vp_W, vp_b, vp_g, vp_be, att_W1, att_b1, att_g, att_be, att_W2, att_b2, geo_W1, geo_b1, geo_g, geo_be, geo_W2, geo_b2, fus_W, fus_b, fus_g, fus_be):
    raise NotImplementedError("write your pallas kernel here")


# speedup vs baseline: 3.6099x; 3.6099x over previous
import functools

import jax
import jax.numpy as jnp
from jax.experimental import pallas as pl

S = 512
CHUNK = 1024


def _ln(x, g, b, eps=1e-5):
    mu = jnp.mean(x, axis=-1, keepdims=True)
    var = jnp.mean((x - mu) ** 2, axis=-1, keepdims=True)
    return (x - mu) / jnp.sqrt(var + eps) * g + b


def _stage_a(pf_ref, vf_ref, ids_ref, mask_ref,
             pp_W, pp_b, pp_g, pp_be, vp_W, vp_b, vp_g, vp_be,
             att_W1, att_b1, att_g, att_be, att_W2, att_b2,
             geo_W1, geo_b1, geo_g, geo_be, geo_W2, geo_b2,
             pp_out, vp_out, geo_out, attn_out, sgeo_out, saux_out):
    c = pl.program_id(1)
    x = pf_ref[...]
    pp = jax.nn.relu(_ln(x @ pp_W[...] + pp_b[...], pp_g[...], pp_be[...]))
    v = vf_ref[...]
    vp = jax.nn.relu(_ln(v @ vp_W[...] + vp_b[...], vp_g[...], vp_be[...]))
    vp = vp * mask_ref[...]
    cat = jnp.concatenate([pp, vp], axis=1)
    h = jax.nn.relu(_ln(cat @ att_W1[...] + att_b1[...], att_g[...], att_be[...]))
    attn = jax.nn.sigmoid(h @ att_W2[...] + att_b2[...])
    g1 = jax.nn.relu(_ln(cat @ geo_W1[...] + geo_b1[...], geo_g[...], geo_be[...]))
    geo = g1 @ geo_W2[...] + geo_b2[...]

    pp_out[...] = pp
    vp_out[...] = vp
    geo_out[...] = geo
    attn_out[...] = attn

    ids = ids_ref[...]
    onehot = (ids == jax.lax.broadcasted_iota(jnp.int32, (ids.shape[0], S), 1))
    onehot = onehot.astype(jnp.float32)
    psgeo = jax.lax.dot_general(onehot, geo, (((0,), (0,)), ((), ())),
                                preferred_element_type=jnp.float32)
    aux = jnp.concatenate([attn, jnp.ones_like(attn)], axis=1)
    psaux = jax.lax.dot_general(onehot, aux, (((0,), (0,)), ((), ())),
                                preferred_element_type=jnp.float32)

    @pl.when(c == 0)
    def _():
        sgeo_out[...] = psgeo
        saux_out[...] = psaux

    @pl.when(c != 0)
    def _():
        sgeo_out[...] += psgeo
        saux_out[...] += psaux


def _stage_b(sgeo_ref, saux_ref, pp_ref, vp_ref, geo_ref, attn_ref, ids_ref,
             fus_W, fus_b, fus_g, fus_be, out_ref):
    saux = saux_ref[...]
    cnt = jnp.maximum(saux[:, 1:2], 1.0)
    mgeo = sgeo_ref[...] / cnt
    mattn = saux[:, 0:1] / cnt

    ids = ids_ref[...]
    onehot = (ids == jax.lax.broadcasted_iota(jnp.int32, (ids.shape[0], S), 1))
    onehot = onehot.astype(jnp.float32)
    mg = jnp.dot(onehot, mgeo, preferred_element_type=jnp.float32)
    ma = jnp.dot(onehot, mattn, preferred_element_type=jnp.float32)

    geo = geo_ref[...]
    num = jnp.sum(geo * mg, axis=1, keepdims=True)
    dn = jnp.sqrt(jnp.sum(geo * geo, axis=1, keepdims=True)) * \
         jnp.sqrt(jnp.sum(mg * mg, axis=1, keepdims=True))
    dn = jnp.maximum(dn, 1e-8)
    sim = num / dn
    refined = ma + 0.2 * (attn_ref[...] - ma) * sim

    attended = vp_ref[...] * refined
    comb = jnp.concatenate([pp_ref[...], attended], axis=1)
    o = comb @ fus_W[...] + fus_b[...]
    out_ref[...] = _ln(o, fus_g[...], fus_be[...])


@jax.jit
def kernel(point_features, view_features, superpoint_ids, valid_mask,
           pp_W, pp_b, pp_g, pp_be, vp_W, vp_b, vp_g, vp_be,
           att_W1, att_b1, att_g, att_be, att_W2, att_b2,
           geo_W1, geo_b1, geo_g, geo_be, geo_W2, geo_b2,
           fus_W, fus_b, fus_g, fus_be):
    B, NP, PD = point_features.shape
    VD = view_features.shape[-1]
    H = pp_W.shape[1]
    FD = fus_W.shape[1]
    C = NP // CHUNK
    Bn = B * NP

    pf = point_features.reshape(Bn, PD)
    vf = view_features.reshape(Bn, VD)
    ids = superpoint_ids.astype(jnp.int32).reshape(Bn, 1)
    mask = valid_mask.astype(jnp.float32).reshape(Bn, 1)

    def r2(x):
        return x.reshape(1, -1)

    row_spec = lambda w: pl.BlockSpec((CHUNK, w), lambda b, c: (b * C + c, 0))
    full_spec = lambda shp: pl.BlockSpec(shp, lambda b, c: tuple(0 for _ in shp))

    wspecs_a = [full_spec(s) for s in
                [(PD, H), (1, H), (1, H), (1, H),
                 (VD, H), (1, H), (1, H), (1, H),
                 (2 * H, H), (1, H), (1, H), (1, H), (H, 1), (1, 1),
                 (2 * H, H), (1, H), (1, H), (1, H), (H, H), (1, H)]]

    pp_o, vp_o, geo_o, attn_o, sgeo, saux = pl.pallas_call(
        _stage_a,
        grid=(B, C),
        in_specs=[row_spec(PD), row_spec(VD), row_spec(1), row_spec(1)] + wspecs_a,
        out_specs=[
            row_spec(H), row_spec(H), row_spec(H), row_spec(1),
            pl.BlockSpec((S, H), lambda b, c: (b, 0)),
            pl.BlockSpec((S, 2), lambda b, c: (b, 0)),
        ],
        out_shape=[
            jax.ShapeDtypeStruct((Bn, H), jnp.float32),
            jax.ShapeDtypeStruct((Bn, H), jnp.float32),
            jax.ShapeDtypeStruct((Bn, H), jnp.float32),
            jax.ShapeDtypeStruct((Bn, 1), jnp.float32),
            jax.ShapeDtypeStruct((B * S, H), jnp.float32),
            jax.ShapeDtypeStruct((B * S, 2), jnp.float32),
        ],
    )(pf, vf, ids, mask,
      pp_W, r2(pp_b), r2(pp_g), r2(pp_be), vp_W, r2(vp_b), r2(vp_g), r2(vp_be),
      att_W1, r2(att_b1), r2(att_g), r2(att_be), att_W2, r2(att_b2),
      geo_W1, r2(geo_b1), r2(geo_g), r2(geo_be), geo_W2, r2(geo_b2))

    out = pl.pallas_call(
        _stage_b,
        grid=(B, C),
        in_specs=[
            pl.BlockSpec((S, H), lambda b, c: (b, 0)),
            pl.BlockSpec((S, 2), lambda b, c: (b, 0)),
            row_spec(H), row_spec(H), row_spec(H), row_spec(1), row_spec(1),
            full_spec((2 * H, FD)), full_spec((1, FD)),
            full_spec((1, FD)), full_spec((1, FD)),
        ],
        out_specs=row_spec(FD),
        out_shape=jax.ShapeDtypeStruct((Bn, FD), jnp.float32),
    )(sgeo, saux, pp_o, vp_o, geo_o, attn_o, ids,
      fus_W, r2(fus_b), r2(fus_g), r2(fus_be))

    return out.reshape(B, NP, FD)


# speedup vs baseline: 3.6175x; 1.0021x over previous
import functools

import jax
import jax.numpy as jnp
from jax.experimental import pallas as pl

S = 512
CHUNK = 1024


def _ln(x, g, b, eps=1e-5):
    mu = jnp.mean(x, axis=-1, keepdims=True)
    var = jnp.mean((x - mu) ** 2, axis=-1, keepdims=True)
    return (x - mu) / jnp.sqrt(var + eps) * g + b


def _mm(a, b):
    return jax.lax.dot_general(
        a.astype(jnp.bfloat16), b.astype(jnp.bfloat16),
        (((1,), (0,)), ((), ())), preferred_element_type=jnp.float32)


def _mmT(a, b):
    return jax.lax.dot_general(
        a.astype(jnp.bfloat16), b.astype(jnp.bfloat16),
        (((0,), (0,)), ((), ())), preferred_element_type=jnp.float32)


def _stage_a(pf_ref, vf_ref, ids_ref, mask_ref,
             pp_W, pp_b, pp_g, pp_be, vp_W, vp_b, vp_g, vp_be,
             att_W1, att_b1, att_g, att_be, att_W2, att_b2,
             geo_W1, geo_b1, geo_g, geo_be, geo_W2, geo_b2,
             pp_out, vp_out, geo_out, attn_out, sgeo_out, saux_out):
    c = pl.program_id(1)
    x = pf_ref[...]
    pp = jax.nn.relu(_ln(_mm(x, pp_W[...]) + pp_b[...], pp_g[...], pp_be[...]))
    v = vf_ref[...]
    vp = jax.nn.relu(_ln(_mm(v, vp_W[...]) + vp_b[...], vp_g[...], vp_be[...]))
    vp = vp * mask_ref[...]
    cat = jnp.concatenate([pp, vp], axis=1)
    h = jax.nn.relu(_ln(_mm(cat, att_W1[...]) + att_b1[...], att_g[...], att_be[...]))
    attn = jax.nn.sigmoid(_mm(h, att_W2[...]) + att_b2[...])
    g1 = jax.nn.relu(_ln(_mm(cat, geo_W1[...]) + geo_b1[...], geo_g[...], geo_be[...]))
    geo = _mm(g1, geo_W2[...]) + geo_b2[...]

    pp_out[...] = pp
    vp_out[...] = vp
    geo_out[...] = geo
    attn_out[...] = attn

    ids = ids_ref[...]
    onehot = (ids == jax.lax.broadcasted_iota(jnp.int32, (ids.shape[0], S), 1))
    onehot = onehot.astype(jnp.float32)
    psgeo = _mmT(onehot, geo)
    aux = jnp.concatenate([attn, jnp.ones_like(attn)], axis=1)
    psaux = _mmT(onehot, aux)

    @pl.when(c == 0)
    def _():
        sgeo_out[...] = psgeo
        saux_out[...] = psaux

    @pl.when(c != 0)
    def _():
        sgeo_out[...] += psgeo
        saux_out[...] += psaux


def _stage_b(sgeo_ref, saux_ref, pp_ref, vp_ref, geo_ref, attn_ref, ids_ref,
             fus_W, fus_b, fus_g, fus_be, out_ref):
    saux = saux_ref[...]
    cnt = jnp.maximum(saux[:, 1:2], 1.0)
    mgeo = sgeo_ref[...] / cnt
    mattn = saux[:, 0:1] / cnt

    ids = ids_ref[...]
    onehot = (ids == jax.lax.broadcasted_iota(jnp.int32, (ids.shape[0], S), 1))
    onehot = onehot.astype(jnp.float32)
    mg = _mm(onehot, mgeo)
    ma = _mm(onehot, mattn)

    geo = geo_ref[...]
    num = jnp.sum(geo * mg, axis=1, keepdims=True)
    dn = jnp.sqrt(jnp.sum(geo * geo, axis=1, keepdims=True)) * \
         jnp.sqrt(jnp.sum(mg * mg, axis=1, keepdims=True))
    dn = jnp.maximum(dn, 1e-8)
    sim = num / dn
    refined = ma + 0.2 * (attn_ref[...] - ma) * sim

    attended = vp_ref[...] * refined
    comb = jnp.concatenate([pp_ref[...], attended], axis=1)
    o = _mm(comb, fus_W[...]) + fus_b[...]
    out_ref[...] = _ln(o, fus_g[...], fus_be[...])


@jax.jit
def kernel(point_features, view_features, superpoint_ids, valid_mask,
           pp_W, pp_b, pp_g, pp_be, vp_W, vp_b, vp_g, vp_be,
           att_W1, att_b1, att_g, att_be, att_W2, att_b2,
           geo_W1, geo_b1, geo_g, geo_be, geo_W2, geo_b2,
           fus_W, fus_b, fus_g, fus_be):
    B, NP, PD = point_features.shape
    VD = view_features.shape[-1]
    H = pp_W.shape[1]
    FD = fus_W.shape[1]
    C = NP // CHUNK
    Bn = B * NP

    pf = point_features.reshape(Bn, PD)
    vf = view_features.reshape(Bn, VD)
    ids = superpoint_ids.astype(jnp.int32).reshape(Bn, 1)
    mask = valid_mask.astype(jnp.float32).reshape(Bn, 1)

    def r2(x):
        return x.reshape(1, -1)

    row_spec = lambda w: pl.BlockSpec((CHUNK, w), lambda b, c: (b * C + c, 0))
    full_spec = lambda shp: pl.BlockSpec(shp, lambda b, c: tuple(0 for _ in shp))

    wspecs_a = [full_spec(s) for s in
                [(PD, H), (1, H), (1, H), (1, H),
                 (VD, H), (1, H), (1, H), (1, H),
                 (2 * H, H), (1, H), (1, H), (1, H), (H, 1), (1, 1),
                 (2 * H, H), (1, H), (1, H), (1, H), (H, H), (1, H)]]

    pp_o, vp_o, geo_o, attn_o, sgeo, saux = pl.pallas_call(
        _stage_a,
        grid=(B, C),
        in_specs=[row_spec(PD), row_spec(VD), row_spec(1), row_spec(1)] + wspecs_a,
        out_specs=[
            row_spec(H), row_spec(H), row_spec(H), row_spec(1),
            pl.BlockSpec((S, H), lambda b, c: (b, 0)),
            pl.BlockSpec((S, 2), lambda b, c: (b, 0)),
        ],
        out_shape=[
            jax.ShapeDtypeStruct((Bn, H), jnp.float32),
            jax.ShapeDtypeStruct((Bn, H), jnp.float32),
            jax.ShapeDtypeStruct((Bn, H), jnp.float32),
            jax.ShapeDtypeStruct((Bn, 1), jnp.float32),
            jax.ShapeDtypeStruct((B * S, H), jnp.float32),
            jax.ShapeDtypeStruct((B * S, 2), jnp.float32),
        ],
    )(pf, vf, ids, mask,
      pp_W, r2(pp_b), r2(pp_g), r2(pp_be), vp_W, r2(vp_b), r2(vp_g), r2(vp_be),
      att_W1, r2(att_b1), r2(att_g), r2(att_be), att_W2, r2(att_b2),
      geo_W1, r2(geo_b1), r2(geo_g), r2(geo_be), geo_W2, r2(geo_b2))

    out = pl.pallas_call(
        _stage_b,
        grid=(B, C),
        in_specs=[
            pl.BlockSpec((S, H), lambda b, c: (b, 0)),
            pl.BlockSpec((S, 2), lambda b, c: (b, 0)),
            row_spec(H), row_spec(H), row_spec(H), row_spec(1), row_spec(1),
            full_spec((2 * H, FD)), full_spec((1, FD)),
            full_spec((1, FD)), full_spec((1, FD)),
        ],
        out_specs=row_spec(FD),
        out_shape=jax.ShapeDtypeStruct((Bn, FD), jnp.float32),
    )(sgeo, saux, pp_o, vp_o, geo_o, attn_o, ids,
      fus_W, r2(fus_b), r2(fus_g), r2(fus_be))

    return out.reshape(B, NP, FD)


# speedup vs baseline: 3.6492x; 1.0088x over previous
import jax
import jax.numpy as jnp
from jax.experimental import pallas as pl

S = 512
CHUNK = 2048


def _ln(x, g, b, eps=1e-5):
    n = x.shape[-1]
    s1 = jnp.sum(x, axis=-1, keepdims=True)
    s2 = jnp.sum(x * x, axis=-1, keepdims=True)
    mu = s1 * (1.0 / n)
    var = s2 * (1.0 / n) - mu * mu
    inv = jax.lax.rsqrt(var + eps)
    return (x - mu) * inv * g + b


def _mm(a, b):
    return jax.lax.dot_general(
        a.astype(jnp.bfloat16), b, (((1,), (0,)), ((), ())),
        preferred_element_type=jnp.float32)


def _stage_a(pf_ref, vf_ref, ids_ref, mask_ref,
             pp_W, pp_b, pp_g, pp_be, vp_W, vp_b, vp_g, vp_be,
             att_W1, att_b1, att_g, att_be, att_W2, att_b2,
             geo_W1, geo_b1, geo_g, geo_be, geo_W2, geo_b2,
             pp_out, vp_out, geo_out, attn_out, sgeo_out, saux_out):
    c = pl.program_id(1)
    x = pf_ref[...]
    pp = jax.nn.relu(_ln(_mm(x, pp_W[...]) + pp_b[...], pp_g[...], pp_be[...]))
    v = vf_ref[...]
    vp = jax.nn.relu(_ln(_mm(v, vp_W[...]) + vp_b[...], vp_g[...], vp_be[...]))
    vp = vp * mask_ref[...]
    cat = jnp.concatenate([pp, vp], axis=1)
    h = jax.nn.relu(_ln(_mm(cat, att_W1[...]) + att_b1[...], att_g[...], att_be[...]))
    attn = jax.nn.sigmoid(_mm(h, att_W2[...]) + att_b2[...])
    g1 = jax.nn.relu(_ln(_mm(cat, geo_W1[...]) + geo_b1[...], geo_g[...], geo_be[...]))
    geo = _mm(g1, geo_W2[...]) + geo_b2[...]

    pp_out[...] = pp
    vp_out[...] = vp
    geo_out[...] = geo
    attn_out[...] = attn

    ids = ids_ref[...]
    iota = jax.lax.broadcasted_iota(jnp.int32, (ids.shape[0], S), 1)
    eq = ids == iota
    oh_bf = eq.astype(jnp.bfloat16)
    oh_f = eq.astype(jnp.float32)
    psgeo = jax.lax.dot_general(
        oh_bf, geo.astype(jnp.bfloat16), (((0,), (0,)), ((), ())),
        preferred_element_type=jnp.float32)
    aux = jnp.concatenate([attn, jnp.ones_like(attn)], axis=1)
    psaux = jax.lax.dot_general(oh_f, aux, (((0,), (0,)), ((), ())),
                                preferred_element_type=jnp.float32)

    @pl.when(c == 0)
    def _():
        sgeo_out[...] = psgeo
        saux_out[...] = psaux

    @pl.when(c != 0)
    def _():
        sgeo_out[...] += psgeo
        saux_out[...] += psaux


def _stage_b(sgeo_ref, saux_ref, pp_ref, vp_ref, geo_ref, attn_ref, ids_ref,
             fus_W, fus_b, fus_g, fus_be, out_ref):
    ids = ids_ref[...]
    iota = jax.lax.broadcasted_iota(jnp.int32, (ids.shape[0], S), 1)
    eq = ids == iota
    oh_bf = eq.astype(jnp.bfloat16)
    oh_f = eq.astype(jnp.float32)
    mg = jax.lax.dot_general(
        oh_bf, sgeo_ref[...].astype(jnp.bfloat16), (((1,), (0,)), ((), ())),
        preferred_element_type=jnp.float32)
    aux = jnp.dot(oh_f, saux_ref[...], preferred_element_type=jnp.float32)
    ma = aux[:, 0:1] / jnp.maximum(aux[:, 1:2], 1.0)

    geo = geo_ref[...]
    num = jnp.sum(geo * mg, axis=1, keepdims=True)
    dn = jnp.sqrt(jnp.sum(geo * geo, axis=1, keepdims=True)) * \
         jnp.sqrt(jnp.sum(mg * mg, axis=1, keepdims=True))
    dn = jnp.maximum(dn, 1e-8)
    sim = num / dn
    refined = ma + 0.2 * (attn_ref[...] - ma) * sim

    attended = vp_ref[...] * refined
    comb = jnp.concatenate([pp_ref[...], attended], axis=1)
    o = _mm(comb, fus_W[...]) + fus_b[...]
    out_ref[...] = _ln(o, fus_g[...], fus_be[...])


@jax.jit
def kernel(point_features, view_features, superpoint_ids, valid_mask,
           pp_W, pp_b, pp_g, pp_be, vp_W, vp_b, vp_g, vp_be,
           att_W1, att_b1, att_g, att_be, att_W2, att_b2,
           geo_W1, geo_b1, geo_g, geo_be, geo_W2, geo_b2,
           fus_W, fus_b, fus_g, fus_be):
    B, NP, PD = point_features.shape
    VD = view_features.shape[-1]
    H = pp_W.shape[1]
    FD = fus_W.shape[1]
    C = NP // CHUNK
    Bn = B * NP

    pf = point_features.reshape(Bn, PD)
    vf = view_features.reshape(Bn, VD)
    ids = superpoint_ids.astype(jnp.int32).reshape(Bn, 1)
    mask = valid_mask.astype(jnp.float32).reshape(Bn, 1)

    def bf(w):
        return w.astype(jnp.bfloat16)

    def r2(x):
        return x.reshape(1, -1)

    row_spec = lambda w: pl.BlockSpec((CHUNK, w), lambda b, c: (b * C + c, 0))
    full_spec = lambda shp: pl.BlockSpec(shp, lambda b, c: tuple(0 for _ in shp))

    wspecs_a = [full_spec(s) for s in
                [(PD, H), (1, H), (1, H), (1, H),
                 (VD, H), (1, H), (1, H), (1, H),
                 (2 * H, H), (1, H), (1, H), (1, H), (H, 1), (1, 1),
                 (2 * H, H), (1, H), (1, H), (1, H), (H, H), (1, H)]]

    pp_o, vp_o, geo_o, attn_o, sgeo, saux = pl.pallas_call(
        _stage_a,
        grid=(B, C),
        in_specs=[row_spec(PD), row_spec(VD), row_spec(1), row_spec(1)] + wspecs_a,
        out_specs=[
            row_spec(H), row_spec(H), row_spec(H), row_spec(1),
            pl.BlockSpec((S, H), lambda b, c: (b, 0)),
            pl.BlockSpec((S, 2), lambda b, c: (b, 0)),
        ],
        out_shape=[
            jax.ShapeDtypeStruct((Bn, H), jnp.float32),
            jax.ShapeDtypeStruct((Bn, H), jnp.float32),
            jax.ShapeDtypeStruct((Bn, H), jnp.float32),
            jax.ShapeDtypeStruct((Bn, 1), jnp.float32),
            jax.ShapeDtypeStruct((B * S, H), jnp.float32),
            jax.ShapeDtypeStruct((B * S, 2), jnp.float32),
        ],
    )(pf, vf, ids, mask,
      bf(pp_W), r2(pp_b), r2(pp_g), r2(pp_be),
      bf(vp_W), r2(vp_b), r2(vp_g), r2(vp_be),
      bf(att_W1), r2(att_b1), r2(att_g), r2(att_be), bf(att_W2), r2(att_b2),
      bf(geo_W1), r2(geo_b1), r2(geo_g), r2(geo_be), bf(geo_W2), r2(geo_b2))

    out = pl.pallas_call(
        _stage_b,
        grid=(B, C),
        in_specs=[
            pl.BlockSpec((S, H), lambda b, c: (b, 0)),
            pl.BlockSpec((S, 2), lambda b, c: (b, 0)),
            row_spec(H), row_spec(H), row_spec(H), row_spec(1), row_spec(1),
            full_spec((2 * H, FD)), full_spec((1, FD)),
            full_spec((1, FD)), full_spec((1, FD)),
        ],
        out_specs=row_spec(FD),
        out_shape=jax.ShapeDtypeStruct((Bn, FD), jnp.float32),
    )(sgeo, saux, pp_o, vp_o, geo_o, attn_o, ids,
      bf(fus_W), r2(fus_b), r2(fus_g), r2(fus_be))

    return out.reshape(B, NP, FD)


# speedup vs baseline: 4.3683x; 1.1971x over previous
import jax
import jax.numpy as jnp
from jax.experimental import pallas as pl
from jax.experimental.pallas import tpu as pltpu

S = 512
SUB = 1024


def _ln(x, g, b, eps=1e-5):
    n = x.shape[-1]
    s1 = jnp.sum(x, axis=-1, keepdims=True)
    s2 = jnp.sum(x * x, axis=-1, keepdims=True)
    mu = s1 * (1.0 / n)
    var = s2 * (1.0 / n) - mu * mu
    inv = jax.lax.rsqrt(var + eps)
    return (x - mu) * inv * g + b


def _mm(a, b):
    return jax.lax.dot_general(
        a.astype(jnp.bfloat16), b, (((1,), (0,)), ((), ())),
        preferred_element_type=jnp.float32)


def _fused(pf_ref, vf_ref, ids_ref, mask_ref,
           pp_W, pp_b, pp_g, pp_be, vp_W, vp_b, vp_g, vp_be,
           att_W1, att_b1, att_g, att_be, att_W2, att_b2,
           geo_W1, geo_b1, geo_g, geo_be, geo_W2, geo_b2,
           fus_W, fus_b, fus_g, fus_be,
           out_ref,
           pp_s, vp_s, geo_s, attn_s, sgeo_s, saux_s):
    npts = pf_ref.shape[0]
    nsub = npts // SUB

    sgeo_acc = None
    saux_acc = None
    for i in range(nsub):
        sl = pl.ds(i * SUB, SUB)
        x = pf_ref[sl, :]
        pp = jax.nn.relu(_ln(_mm(x, pp_W[...]) + pp_b[...], pp_g[...], pp_be[...]))
        v = vf_ref[sl, :]
        vp = jax.nn.relu(_ln(_mm(v, vp_W[...]) + vp_b[...], vp_g[...], vp_be[...]))
        vp = vp * mask_ref[sl, :]
        cat = jnp.concatenate([pp, vp], axis=1)
        h = jax.nn.relu(_ln(_mm(cat, att_W1[...]) + att_b1[...], att_g[...], att_be[...]))
        attn = jax.nn.sigmoid(_mm(h, att_W2[...]) + att_b2[...])
        g1 = jax.nn.relu(_ln(_mm(cat, geo_W1[...]) + geo_b1[...], geo_g[...], geo_be[...]))
        geo = _mm(g1, geo_W2[...]) + geo_b2[...]

        pp_s[sl, :] = pp.astype(jnp.bfloat16)
        vp_s[sl, :] = vp
        geo_s[sl, :] = geo
        attn_s[sl, :] = attn

        ids = ids_ref[sl, :]
        iota = jax.lax.broadcasted_iota(jnp.int32, (SUB, S), 1)
        eq = ids == iota
        psgeo = jax.lax.dot_general(
            eq.astype(jnp.bfloat16), geo.astype(jnp.bfloat16),
            (((0,), (0,)), ((), ())), preferred_element_type=jnp.float32)
        aux = jnp.concatenate([attn, jnp.ones_like(attn)], axis=1)
        psaux = jax.lax.dot_general(
            eq.astype(jnp.float32), aux, (((0,), (0,)), ((), ())),
            preferred_element_type=jnp.float32)
        sgeo_acc = psgeo if i == 0 else sgeo_acc + psgeo
        saux_acc = psaux if i == 0 else saux_acc + psaux

    sgeo_s[...] = sgeo_acc.astype(jnp.bfloat16)
    saux_s[...] = saux_acc

    for i in range(nsub):
        sl = pl.ds(i * SUB, SUB)
        ids = ids_ref[sl, :]
        iota = jax.lax.broadcasted_iota(jnp.int32, (SUB, S), 1)
        eq = ids == iota
        oh_bf = eq.astype(jnp.bfloat16)
        mg = jax.lax.dot_general(
            oh_bf, sgeo_s[...], (((1,), (0,)), ((), ())),
            preferred_element_type=jnp.float32)
        aux = jnp.dot(eq.astype(jnp.float32), saux_s[...],
                      preferred_element_type=jnp.float32)
        ma = aux[:, 0:1] / jnp.maximum(aux[:, 1:2], 1.0)

        geo = geo_s[sl, :]
        num = jnp.sum(geo * mg, axis=1, keepdims=True)
        dn = jnp.sqrt(jnp.sum(geo * geo, axis=1, keepdims=True)) * \
             jnp.sqrt(jnp.sum(mg * mg, axis=1, keepdims=True))
        dn = jnp.maximum(dn, 1e-8)
        sim = num / dn
        refined = ma + 0.2 * (attn_s[sl, :] - ma) * sim

        attended = vp_s[sl, :] * refined
        comb = jnp.concatenate([pp_s[sl, :].astype(jnp.float32), attended], axis=1)
        o = _mm(comb, fus_W[...]) + fus_b[...]
        out_ref[sl, :] = _ln(o, fus_g[...], fus_be[...])


@jax.jit
def kernel(point_features, view_features, superpoint_ids, valid_mask,
           pp_W, pp_b, pp_g, pp_be, vp_W, vp_b, vp_g, vp_be,
           att_W1, att_b1, att_g, att_be, att_W2, att_b2,
           geo_W1, geo_b1, geo_g, geo_be, geo_W2, geo_b2,
           fus_W, fus_b, fus_g, fus_be):
    B, NP, PD = point_features.shape
    VD = view_features.shape[-1]
    H = pp_W.shape[1]
    FD = fus_W.shape[1]
    Bn = B * NP

    pf = point_features.reshape(Bn, PD)
    vf = view_features.reshape(Bn, VD)
    ids = superpoint_ids.astype(jnp.int32).reshape(Bn, 1)
    mask = valid_mask.astype(jnp.float32).reshape(Bn, 1)

    def bf(w):
        return w.astype(jnp.bfloat16)

    def r2(x):
        return x.reshape(1, -1)

    row_spec = lambda w: pl.BlockSpec((NP, w), lambda b: (b, 0))
    full_spec = lambda shp: pl.BlockSpec(shp, lambda b: tuple(0 for _ in shp))

    wspecs = [full_spec(s) for s in
              [(PD, H), (1, H), (1, H), (1, H),
               (VD, H), (1, H), (1, H), (1, H),
               (2 * H, H), (1, H), (1, H), (1, H), (H, 1), (1, 1),
               (2 * H, H), (1, H), (1, H), (1, H), (H, H), (1, H),
               (2 * H, FD), (1, FD), (1, FD), (1, FD)]]

    out = pl.pallas_call(
        _fused,
        grid=(B,),
        in_specs=[row_spec(PD), row_spec(VD), row_spec(1), row_spec(1)] + wspecs,
        out_specs=row_spec(FD),
        out_shape=jax.ShapeDtypeStruct((Bn, FD), jnp.float32),
        scratch_shapes=[
            pltpu.VMEM((NP, H), jnp.bfloat16),
            pltpu.VMEM((NP, H), jnp.float32),
            pltpu.VMEM((NP, H), jnp.float32),
            pltpu.VMEM((NP, 1), jnp.float32),
            pltpu.VMEM((S, H), jnp.bfloat16),
            pltpu.VMEM((S, 2), jnp.float32),
        ],
    )(pf, vf, ids, mask,
      bf(pp_W), r2(pp_b), r2(pp_g), r2(pp_be),
      bf(vp_W), r2(vp_b), r2(vp_g), r2(vp_be),
      bf(att_W1), r2(att_b1), r2(att_g), r2(att_be), bf(att_W2), r2(att_b2),
      bf(geo_W1), r2(geo_b1), r2(geo_g), r2(geo_be), bf(geo_W2), r2(geo_b2),
      bf(fus_W), r2(fus_b), r2(fus_g), r2(fus_be))

    return out.reshape(B, NP, FD)
